# Initial kernel scaffold; baseline (speedup 1.0000x reference)
import jax
import jax.numpy as jnp
from jax.experimental import pallas as pl


def kernel(x, W):
    raise NotImplementedError("write your pallas kernel here")


# speedup vs baseline: 2.3997x; 2.3997x over previous
import functools

import jax
import jax.numpy as jnp
from jax.experimental import pallas as pl

_BLK = 2048


def _router_kernel(x_ref, w_ref, idx_ref, wts_ref, colsum_ref, aux_ref,
                   *, n_tokens, n_experts):
    i = pl.program_id(0)
    logits = jax.lax.dot_general(
        x_ref[...], w_ref[...],
        dimension_numbers=(((1,), (1,)), ((), ())),
        preferred_element_type=jnp.float32)
    r, e = logits.shape
    iota = jax.lax.broadcasted_iota(jnp.int32, (r, e), 1)

    m1 = jnp.max(logits, axis=1, keepdims=True)
    idx1 = jnp.min(jnp.where(logits == m1, iota, n_experts), axis=1,
                   keepdims=True)
    masked = jnp.where(iota == idx1, -jnp.inf, logits)
    m2 = jnp.max(masked, axis=1, keepdims=True)
    idx2 = jnp.min(jnp.where(masked == m2, iota, n_experts), axis=1,
                   keepdims=True)

    ex = jnp.exp(logits - m1)
    s = jnp.sum(ex, axis=1, keepdims=True)
    probs = ex / s
    e2 = jnp.exp(m2 - m1)
    w1 = 1.0 / (1.0 + e2)
    w2 = e2 / (1.0 + e2)

    pair = jax.lax.broadcasted_iota(jnp.int32, (r, 2), 1)
    idx_ref[...] = jnp.where(pair == 0, idx1, idx2)
    wts_ref[...] = jnp.where(pair == 0, w1, w2)

    csum = jnp.sum(probs, axis=0, keepdims=True)

    @pl.when(i == 0)
    def _init():
        colsum_ref[...] = csum

    @pl.when(i > 0)
    def _acc():
        colsum_ref[...] = colsum_ref[...] + csum

    @pl.when(i == pl.num_programs(0) - 1)
    def _finish():
        usage = colsum_ref[...] * (1.0 / n_tokens) - (1.0 / n_experts)
        aux_ref[...] = jnp.sum(usage * usage, axis=1, keepdims=True)


def kernel(x, W):
    b, s, d = x.shape
    e = W.shape[0]
    n = b * s
    x_flat = x.reshape(n, d)
    blk = _BLK
    grid = n // blk

    idx, wts, _colsum, aux = pl.pallas_call(
        functools.partial(_router_kernel, n_tokens=n, n_experts=e),
        grid=(grid,),
        in_specs=[
            pl.BlockSpec((blk, d), lambda i: (i, 0)),
            pl.BlockSpec((e, d), lambda i: (0, 0)),
        ],
        out_specs=[
            pl.BlockSpec((blk, 2), lambda i: (i, 0)),
            pl.BlockSpec((blk, 2), lambda i: (i, 0)),
            pl.BlockSpec((1, e), lambda i: (0, 0)),
            pl.BlockSpec((1, 1), lambda i: (0, 0)),
        ],
        out_shape=[
            jax.ShapeDtypeStruct((n, 2), jnp.int32),
            jax.ShapeDtypeStruct((n, 2), jnp.float32),
            jax.ShapeDtypeStruct((1, e), jnp.float32),
            jax.ShapeDtypeStruct((1, 1), jnp.float32),
        ],
    )(x_flat, W)

    return (idx.reshape(b, s, 2), wts.reshape(b, s, 2), aux[0, 0])


# speedup vs baseline: 2.5799x; 1.0751x over previous
import functools

import jax
import jax.numpy as jnp
from jax.experimental import pallas as pl

_BLK = 2048


def _router_kernel(x_ref, w_ref, idx_ref, wts_ref, colsum_ref, aux_ref,
                   *, n_tokens, n_experts):
    i = pl.program_id(0)
    logits = jax.lax.dot_general(
        x_ref[...], w_ref[...],
        dimension_numbers=(((1,), (1,)), ((), ())),
        preferred_element_type=jnp.float32)
    r, e = logits.shape
    hi = (jnp.int32(e - 1)
          - jax.lax.broadcasted_iota(jnp.int32, (r, e), 1)).astype(jnp.float32)

    m1 = jnp.max(logits, axis=1, keepdims=True)
    sel1 = jnp.where(logits == m1, hi, jnp.float32(-1.0))
    t1 = jnp.max(sel1, axis=1, keepdims=True)
    masked = jnp.where(sel1 == t1, -jnp.inf, logits)
    m2 = jnp.max(masked, axis=1, keepdims=True)
    sel2 = jnp.where(masked == m2, hi, jnp.float32(-1.0))
    t2 = jnp.max(sel2, axis=1, keepdims=True)
    idx1 = (jnp.float32(e - 1) - t1).astype(jnp.int32)
    idx2 = (jnp.float32(e - 1) - t2).astype(jnp.int32)

    ex = jnp.exp(logits - m1)
    s = jnp.sum(ex, axis=1, keepdims=True)
    probs = ex * (1.0 / s)
    e2 = jnp.exp(m2 - m1)
    w1 = 1.0 / (1.0 + e2)
    w2 = e2 / (1.0 + e2)

    pair = jax.lax.broadcasted_iota(jnp.int32, (r, 2), 1)
    idx_ref[...] = jnp.where(pair == 0, idx1, idx2)
    wts_ref[...] = jnp.where(pair == 0, w1, w2)

    csum = jax.lax.dot_general(
        jnp.ones((1, r), jnp.float32), probs,
        dimension_numbers=(((1,), (0,)), ((), ())),
        preferred_element_type=jnp.float32)

    @pl.when(i == 0)
    def _init():
        colsum_ref[...] = csum

    @pl.when(i > 0)
    def _acc():
        colsum_ref[...] = colsum_ref[...] + csum

    @pl.when(i == pl.num_programs(0) - 1)
    def _finish():
        usage = colsum_ref[...] * (1.0 / n_tokens) - (1.0 / n_experts)
        aux_ref[...] = jnp.sum(usage * usage, axis=1, keepdims=True)


def kernel(x, W):
    b, s, d = x.shape
    e = W.shape[0]
    n = b * s
    x_flat = x.reshape(n, d)
    blk = _BLK
    grid = n // blk

    idx, wts, _colsum, aux = pl.pallas_call(
        functools.partial(_router_kernel, n_tokens=n, n_experts=e),
        grid=(grid,),
        in_specs=[
            pl.BlockSpec((blk, d), lambda i: (i, 0)),
            pl.BlockSpec((e, d), lambda i: (0, 0)),
        ],
        out_specs=[
            pl.BlockSpec((blk, 2), lambda i: (i, 0)),
            pl.BlockSpec((blk, 2), lambda i: (i, 0)),
            pl.BlockSpec((1, e), lambda i: (0, 0)),
            pl.BlockSpec((1, 1), lambda i: (0, 0)),
        ],
        out_shape=[
            jax.ShapeDtypeStruct((n, 2), jnp.int32),
            jax.ShapeDtypeStruct((n, 2), jnp.float32),
            jax.ShapeDtypeStruct((1, e), jnp.float32),
            jax.ShapeDtypeStruct((1, 1), jnp.float32),
        ],
    )(x_flat, W)

    return (idx.reshape(b, s, 2), wts.reshape(b, s, 2), aux[0, 0])


# speedup vs baseline: 2.5835x; 1.0014x over previous
import functools

import jax
import jax.numpy as jnp
from jax.experimental import pallas as pl

_BLK = 2048


def _router_kernel(x_ref, w_ref, idx_ref, wts_ref, colsum_ref, aux_ref,
                   *, n_tokens, n_experts):
    i = pl.program_id(0)
    logits = jax.lax.dot_general(
        x_ref[0], w_ref[...],
        dimension_numbers=(((1,), (1,)), ((), ())),
        preferred_element_type=jnp.float32)
    r, e = logits.shape
    hi = (jnp.int32(e - 1)
          - jax.lax.broadcasted_iota(jnp.int32, (r, e), 1)).astype(jnp.float32)

    m1 = jnp.max(logits, axis=1, keepdims=True)
    sel1 = jnp.where(logits == m1, hi, jnp.float32(-1.0))
    t1 = jnp.max(sel1, axis=1, keepdims=True)
    masked = jnp.where(sel1 == t1, -jnp.inf, logits)
    m2 = jnp.max(masked, axis=1, keepdims=True)
    sel2 = jnp.where(masked == m2, hi, jnp.float32(-1.0))
    t2 = jnp.max(sel2, axis=1, keepdims=True)
    idx1 = (jnp.float32(e - 1) - t1).astype(jnp.int32)
    idx2 = (jnp.float32(e - 1) - t2).astype(jnp.int32)

    ex = jnp.exp(logits - m1)
    s = jnp.sum(ex, axis=1, keepdims=True)
    probs = ex * (1.0 / s)
    e2 = jnp.exp(m2 - m1)
    w1 = 1.0 / (1.0 + e2)
    w2 = e2 / (1.0 + e2)

    pair = jax.lax.broadcasted_iota(jnp.int32, (r, 2), 1)
    idx_ref[0] = jnp.where(pair == 0, idx1, idx2)
    wts_ref[0] = jnp.where(pair == 0, w1, w2)

    csum = jax.lax.dot_general(
        jnp.ones((1, r), jnp.float32), probs,
        dimension_numbers=(((1,), (0,)), ((), ())),
        preferred_element_type=jnp.float32)

    @pl.when(i == 0)
    def _init():
        colsum_ref[...] = csum

    @pl.when(i > 0)
    def _acc():
        colsum_ref[...] = colsum_ref[...] + csum

    @pl.when(i == pl.num_programs(0) - 1)
    def _finish():
        usage = colsum_ref[...] * (1.0 / n_tokens) - (1.0 / n_experts)
        aux_ref[...] = jnp.sum(usage * usage, axis=1, keepdims=True)


def kernel(x, W):
    b, s, d = x.shape
    e = W.shape[0]
    n = b * s
    blk = _BLK
    spb = s // blk
    grid = n // blk

    idx, wts, _colsum, aux = pl.pallas_call(
        functools.partial(_router_kernel, n_tokens=n, n_experts=e),
        grid=(grid,),
        in_specs=[
            pl.BlockSpec((1, blk, d), lambda i: (i // spb, i % spb, 0)),
            pl.BlockSpec((e, d), lambda i: (0, 0)),
        ],
        out_specs=[
            pl.BlockSpec((1, blk, 2), lambda i: (i // spb, i % spb, 0)),
            pl.BlockSpec((1, blk, 2), lambda i: (i // spb, i % spb, 0)),
            pl.BlockSpec((1, e), lambda i: (0, 0)),
            pl.BlockSpec((1, 1), lambda i: (0, 0)),
        ],
        out_shape=[
            jax.ShapeDtypeStruct((b, s, 2), jnp.int32),
            jax.ShapeDtypeStruct((b, s, 2), jnp.float32),
            jax.ShapeDtypeStruct((1, e), jnp.float32),
            jax.ShapeDtypeStruct((1, 1), jnp.float32),
        ],
    )(x, W)

    return (idx, wts, aux[0, 0])


# speedup vs baseline: 5.2892x; 2.0473x over previous
import functools

import jax
import jax.numpy as jnp
from jax.experimental import pallas as pl

_BLK = 4096


def _router_kernel(x_ref, w_ref, idx_ref, wts_ref, colsum_ref, aux_ref,
                   *, n_tokens, n_experts):
    i = pl.program_id(0)
    logits = jax.lax.dot_general(
        w_ref[...], x_ref[0],
        dimension_numbers=(((1,), (1,)), ((), ())),
        preferred_element_type=jnp.float32)
    e, r = logits.shape
    hi = (jnp.int32(e - 1)
          - jax.lax.broadcasted_iota(jnp.int32, (e, r), 0)).astype(jnp.float32)

    m1 = jnp.max(logits, axis=0, keepdims=True)
    sel1 = jnp.where(logits == m1, hi, jnp.float32(-1.0))
    t1 = jnp.max(sel1, axis=0, keepdims=True)
    masked = jnp.where(sel1 == t1, -jnp.inf, logits)
    m2 = jnp.max(masked, axis=0, keepdims=True)
    sel2 = jnp.where(masked == m2, hi, jnp.float32(-1.0))
    t2 = jnp.max(sel2, axis=0, keepdims=True)

    ex = jnp.exp(logits - m1)
    s = jnp.sum(ex, axis=0, keepdims=True)
    probs = ex * (1.0 / s)
    e2 = jnp.exp(m2 - m1)
    w1 = 1.0 / (1.0 + e2)
    w2 = e2 / (1.0 + e2)

    pair = jax.lax.broadcasted_iota(jnp.int32, (2, r), 0)
    fe = jnp.float32(e - 1)
    idx_ref[0] = (fe - jnp.where(pair == 0, t1, t2)).astype(jnp.int32)
    wts_ref[0] = jnp.where(pair == 0, w1, w2)

    csum = jnp.sum(probs, axis=1, keepdims=True)

    @pl.when(i == 0)
    def _init():
        colsum_ref[...] = csum

    @pl.when(i > 0)
    def _acc():
        colsum_ref[...] = colsum_ref[...] + csum

    @pl.when(i == pl.num_programs(0) - 1)
    def _finish():
        usage = colsum_ref[...] * (1.0 / n_tokens) - (1.0 / n_experts)
        aux_ref[...] = jnp.sum(usage * usage, axis=0, keepdims=True)


def kernel(x, W):
    b, s, d = x.shape
    e = W.shape[0]
    n = b * s
    blk = _BLK
    spb = s // blk
    grid = n // blk

    idx, wts, _colsum, aux = pl.pallas_call(
        functools.partial(_router_kernel, n_tokens=n, n_experts=e),
        grid=(grid,),
        in_specs=[
            pl.BlockSpec((1, blk, d), lambda i: (i // spb, i % spb, 0)),
            pl.BlockSpec((e, d), lambda i: (0, 0)),
        ],
        out_specs=[
            pl.BlockSpec((1, 2, blk), lambda i: (i // spb, 0, i % spb)),
            pl.BlockSpec((1, 2, blk), lambda i: (i // spb, 0, i % spb)),
            pl.BlockSpec((e, 1), lambda i: (0, 0)),
            pl.BlockSpec((1, 1), lambda i: (0, 0)),
        ],
        out_shape=[
            jax.ShapeDtypeStruct((b, 2, s), jnp.int32),
            jax.ShapeDtypeStruct((b, 2, s), jnp.float32),
            jax.ShapeDtypeStruct((e, 1), jnp.float32),
            jax.ShapeDtypeStruct((1, 1), jnp.float32),
        ],
    )(x, W)

    return (jnp.swapaxes(idx, 1, 2), jnp.swapaxes(wts, 1, 2), aux[0, 0])
